# Initial kernel scaffold; baseline (speedup 1.0000x reference)
import functools

import jax
import jax.numpy as jnp
import numpy as np
from jax import lax
from jax.experimental import pallas as pl
from jax.experimental.pallas import tpu as pltpu
from jax.experimental.pallas import tpu_sc as plsc

N = 10000
E = 160000
F = 128
NRBF = 20
NB = 3
EPS = 1e-8

NC = 2
NS = 16
NW = NC * NS
EW = 5120
E_PAD = NW * EW
CH = 128
NCHUNK = EW // CH
ROWS_PER_TILE = N // NS
DRAIN_CH = 125
NDRAIN = ROWS_PER_TILE // DRAIN_CH

LOG2 = float(np.log(2.0))
F32 = jnp.float32


def _ssp(x):
    return jax.nn.softplus(x) - LOG2


_RD = 2048


def _desc_body(rbfs_ref, cut_ref, vec_ref, dist_ref, wf_ref, bf_ref,
               *out_refs):
    x = jnp.dot(rbfs_ref[...], wf_ref[...], preferred_element_type=F32)
    x = (x + bf_ref[...]) * cut_ref[...]
    dirs = vec_ref[...] / dist_ref[...]
    for b in range(NB):
        for j in range(3):
            out_refs[b * 3 + j][...] = x[:, (b * 3 + j) * F:(b * 3 + j + 1) * F]
    dirs_ref = out_refs[9]
    for k in range(3):
        dirs_ref[k, :, :] = jnp.broadcast_to(dirs[:, k:k + 1], (_RD, 16))


def _desc_call(rbfs_p, cut_p, vec_p, dist_p, wf, bf2):
    grid = (E_PAD // _RD,)
    d_specs = [pl.BlockSpec((_RD, F), lambda i: (i, 0)) for _ in range(9)]
    return pl.pallas_call(
        _desc_body,
        grid=grid,
        in_specs=[
            pl.BlockSpec((_RD, NRBF), lambda i: (i, 0)),
            pl.BlockSpec((_RD, 1), lambda i: (i, 0)),
            pl.BlockSpec((_RD, 3), lambda i: (i, 0)),
            pl.BlockSpec((_RD, 1), lambda i: (i, 0)),
            pl.BlockSpec((NRBF, NB * 3 * F), lambda i: (0, 0)),
            pl.BlockSpec((1, NB * 3 * F), lambda i: (0, 0)),
        ],
        out_specs=d_specs + [pl.BlockSpec((3, _RD, 16), lambda i: (0, i, 0))],
        out_shape=[jax.ShapeDtypeStruct((E_PAD, F), F32) for _ in range(9)]
        + [jax.ShapeDtypeStruct((3, E_PAD, 16), F32)],
    )(rbfs_p, cut_p, vec_p, dist_p, wf, bf2)


_RN = 1000


def _dense_body(q_ref, w1_ref, b1_ref, w2_ref, b2_ref, x0_ref, x1_ref, x2_ref):
    h = _ssp(jnp.dot(q_ref[...], w1_ref[...], preferred_element_type=F32)
             + b1_ref[...])
    x = jnp.dot(h, w2_ref[...], preferred_element_type=F32) + b2_ref[...]
    x0_ref[...] = x[:, :F]
    x1_ref[...] = x[:, F:2 * F]
    x2_ref[...] = x[:, 2 * F:]


def _dense_call(q, w1, b1, w2, b2):
    grid = (N // _RN,)
    return pl.pallas_call(
        _dense_body,
        grid=grid,
        in_specs=[
            pl.BlockSpec((_RN, F), lambda i: (i, 0)),
            pl.BlockSpec((F, F), lambda i: (0, 0)),
            pl.BlockSpec((1, F), lambda i: (0, 0)),
            pl.BlockSpec((F, 3 * F), lambda i: (0, 0)),
            pl.BlockSpec((1, 3 * F), lambda i: (0, 0)),
        ],
        out_specs=[pl.BlockSpec((_RN, F), lambda i: (i, 0))] * 3,
        out_shape=[jax.ShapeDtypeStruct((N, F), F32)] * 3,
    )(q, w1, b1, w2, b2)


def _mix_body(has_mu, mu_nat, *refs):
    if has_mu:
        (q_ref, mu0_ref, mu1_ref, mu2_ref, dqp_ref, dmup_ref,
         wmix_ref, wm1_ref, bm1_ref, wm2_ref, bm2_ref) = refs[:11]
        out_refs = refs[11:]
        mu_in = (mu0_ref, mu1_ref, mu2_ref)
    else:
        (q_ref, dqp_ref, dmup_ref,
         wmix_ref, wm1_ref, bm1_ref, wm2_ref, bm2_ref) = refs[:8]
        out_refs = refs[8:]
        mu_in = None
    qn = q_ref[...] + dqp_ref[0] + dqp_ref[1]
    mun, mu_V, mu_W = [], [], []
    for k in range(3):
        m = dmup_ref[0, k] + dmup_ref[1, k]
        if has_mu:
            m = m + mu_in[k][...]
        mun.append(m)
        mm = jnp.dot(m, wmix_ref[...], preferred_element_type=F32)
        mu_V.append(mm[:, :F])
        mu_W.append(mm[:, F:])
    vn2 = mu_V[0] ** 2 + mu_V[1] ** 2 + mu_V[2] ** 2
    ctx = jnp.concatenate([qn, jnp.sqrt(vn2 + EPS)], axis=1)
    h = _ssp(jnp.dot(ctx, wm1_ref[...], preferred_element_type=F32)
             + bm1_ref[...])
    xm = jnp.dot(h, wm2_ref[...], preferred_element_type=F32) + bm2_ref[...]
    dot = mu_V[0] * mu_W[0] + mu_V[1] * mu_W[1] + mu_V[2] * mu_W[2]
    out_refs[0][...] = qn + xm[:, :F] + xm[:, 2 * F:] * dot
    for k in range(3):
        mo = mun[k] + xm[:, F:2 * F] * mu_W[k]
        if mu_nat:
            out_refs[1][:, k, :] = mo
        else:
            out_refs[1 + k][...] = mo


def _mix_call(has_mu, mu_nat, q, mu, dqp, dmup, wmix, wm1, bm1, wm2, bm2):
    grid = (N // _RN,)
    row = pl.BlockSpec((_RN, F), lambda i: (i, 0))
    in_specs = [row]
    args = [q]
    if has_mu:
        in_specs += [row] * 3
        args += list(mu)
    in_specs += [
        pl.BlockSpec((NC, _RN, F), lambda i: (0, i, 0)),
        pl.BlockSpec((NC, 3, _RN, F), lambda i: (0, 0, i, 0)),
        pl.BlockSpec((F, 2 * F), lambda i: (0, 0)),
        pl.BlockSpec((2 * F, F), lambda i: (0, 0)),
        pl.BlockSpec((1, F), lambda i: (0, 0)),
        pl.BlockSpec((F, 3 * F), lambda i: (0, 0)),
        pl.BlockSpec((1, 3 * F), lambda i: (0, 0)),
    ]
    args += [dqp, dmup, wmix, wm1, bm1, wm2, bm2]
    if mu_nat:
        out_specs = [row, pl.BlockSpec((_RN, 3, F), lambda i: (i, 0, 0))]
        out_shape = [jax.ShapeDtypeStruct((N, F), F32),
                     jax.ShapeDtypeStruct((N, 3, F), F32)]
    else:
        out_specs = [row] * 4
        out_shape = [jax.ShapeDtypeStruct((N, F), F32)] * 4
    return pl.pallas_call(
        functools.partial(_mix_body, has_mu, mu_nat),
        grid=grid,
        in_specs=in_specs,
        out_specs=out_specs,
        out_shape=out_shape,
    )(*args)


def _sc_body(b0, *refs):
    if b0:
        (x0, x1, x2, d0, d1, d2, dirsb, idxi_h, idxj_h,
         dqp_o, dmup_o,
         acc, ba, bb, bc, bd, be, dv, zbuf, stg, ixi, ixj) = refs
        mu = None
    else:
        (x0, x1, x2, mu0, mu1, mu2, d0, d1, d2, dirsb, idxi_h, idxj_h,
         dqp_o, dmup_o,
         acc, ba, bb, bc, bd, be, dv, zbuf, stg, ixi, ixj) = refs
        mu = (mu0, mu1, mu2)
    c = lax.axis_index("c")
    s = lax.axis_index("s")
    wid = c * NS + s
    ebase = wid * EW
    rbase = s * ROWS_PER_TILE

    def zz(i, carry):
        r = i // 8
        g = i % 8
        zbuf[r, pl.ds((g * 16).astype(jnp.int32), 16)] = jnp.zeros((16,), F32)
        return carry

    lax.fori_loop(0, DRAIN_CH * 8, zz, 0)

    for p in range(4):
        for d in range(NDRAIN):
            pltpu.sync_copy(zbuf, acc.at[pl.ds(rbase + d * DRAIN_CH, DRAIN_CH)])
        plsc.subcore_barrier()

        def chunk(i, carry, p=p):
            base = ebase + i * CH
            pltpu.sync_copy(idxi_h.at[pl.ds(base, CH)], ixi)
            pltpu.sync_copy(idxj_h.at[pl.ds(base, CH)], ixj)
            if p == 0:
                pltpu.sync_copy(x0.at[ixj], ba)
                pltpu.sync_copy(d0.at[pl.ds(base, CH)], bd)

                def ce(e, cc):
                    for g in range(8):
                        sl = pl.ds(g * 16, 16)
                        ba[e, sl] = ba[e, sl] * bd[e, sl]
                    return cc

                lax.fori_loop(0, CH, ce, 0)
            else:
                k = p - 1
                pltpu.sync_copy(x1.at[ixj], ba)
                pltpu.sync_copy(d1.at[pl.ds(base, CH)], bd)
                pltpu.sync_copy(dirsb.at[k].at[pl.ds(base, CH)], dv)
                if not b0:
                    pltpu.sync_copy(x2.at[ixj], bb)
                    pltpu.sync_copy(mu[k].at[ixj], bc)
                    pltpu.sync_copy(d2.at[pl.ds(base, CH)], be)

                def ce(e, cc):
                    dvv = dv[e, pl.ds(0, 16)]
                    for g in range(8):
                        sl = pl.ds(g * 16, 16)
                        r = bd[e, sl] * ba[e, sl] * dvv
                        if not b0:
                            r = r + be[e, sl] * bb[e, sl] * bc[e, sl]
                        ba[e, sl] = r
                    return cc

                lax.fori_loop(0, CH, ce, 0)
            pltpu.sync_copy(ba, acc.at[ixi], add=True)
            return carry

        lax.fori_loop(0, NCHUNK, chunk, 0)
        plsc.subcore_barrier()

        for d in range(NDRAIN):
            r0 = rbase + d * DRAIN_CH
            pltpu.sync_copy(acc.at[pl.ds(r0, DRAIN_CH)], stg)
            if p == 0:
                pltpu.sync_copy(stg, dqp_o.at[c].at[pl.ds(r0, DRAIN_CH)])
            else:
                pltpu.sync_copy(
                    stg, dmup_o.at[c].at[p - 1].at[pl.ds(r0, DRAIN_CH)])


def _sc_call(b0, x, mu, d, dirsb, idxi_p, idxj_p):
    mesh = plsc.VectorSubcoreMesh(core_axis_name="c", subcore_axis_name="s",
                                  num_cores=NC, num_subcores=NS)
    out_type = (jax.ShapeDtypeStruct((NC, N, F), F32),
                jax.ShapeDtypeStruct((NC, 3, N, F), F32))
    scratch = [
        pltpu.VMEM_SHARED((N, F), F32),
        pltpu.VMEM((CH, F), F32),
        pltpu.VMEM((CH, F), F32),
        pltpu.VMEM((CH, F), F32),
        pltpu.VMEM((CH, F), F32),
        pltpu.VMEM((CH, F), F32),
        pltpu.VMEM((CH, 16), F32),
        pltpu.VMEM((DRAIN_CH, F), F32),
        pltpu.VMEM((DRAIN_CH, F), F32),
        pltpu.VMEM((CH,), jnp.int32),
        pltpu.VMEM((CH,), jnp.int32),
    ]
    kern = pl.kernel(functools.partial(_sc_body, b0),
                     out_type=out_type, mesh=mesh, scratch_types=scratch)
    if b0:
        return kern(x[0], x[1], x[2], d[0], d[1], d[2], dirsb, idxi_p, idxj_p)
    return kern(x[0], x[1], x[2], mu[0], mu[1], mu[2],
                d[0], d[1], d[2], dirsb, idxi_p, idxj_p)


def kernel(features, rbfs, distances, vectors, cutoff, idx_i, idx_j,
           Wf, bf, Wi1, bi1, Wi2, bi2, Wmix, Wm1, bm1, Wm2, bm2):
    pad = E_PAD - E
    rbfs_p = jnp.pad(rbfs, ((0, pad), (0, 0)))
    cut_p = jnp.pad(cutoff, (0, pad)).reshape(E_PAD, 1)
    vec_p = jnp.pad(vectors, ((0, pad), (0, 0)))
    dist_p = jnp.pad(distances, (0, pad), constant_values=1.0).reshape(E_PAD, 1)
    idxi_p = jnp.pad(idx_i, (0, pad))
    idxj_p = jnp.pad(idx_j, (0, pad))

    desc_out = _desc_call(rbfs_p, cut_p, vec_p, dist_p, Wf, bf.reshape(1, -1))
    dtabs = [desc_out[b * 3:(b + 1) * 3] for b in range(NB)]
    dirsb = desc_out[9]

    q = features
    mu = None
    mu_nat = None
    for b in range(NB):
        x = _dense_call(q, Wi1[b], bi1[b].reshape(1, F),
                        Wi2[b], bi2[b].reshape(1, 3 * F))
        dqp, dmup = _sc_call(b == 0, x, mu, dtabs[b], dirsb, idxi_p, idxj_p)
        last = b == NB - 1
        outs = _mix_call(b != 0, last, q, mu, dqp, dmup,
                         Wmix[b], Wm1[b], bm1[b].reshape(1, F),
                         Wm2[b], bm2[b].reshape(1, 3 * F))
        if last:
            q, mu_nat = outs
        else:
            q, mu = outs[0], outs[1:]
    return q, mu_nat


# speedup vs baseline: 5.2500x; 5.2500x over previous
import functools

import jax
import jax.numpy as jnp
import numpy as np
from jax import lax
from jax.experimental import pallas as pl
from jax.experimental.pallas import tpu as pltpu
from jax.experimental.pallas import tpu_sc as plsc

N = 10000
E = 160000
F = 128
NRBF = 20
NB = 3
EPS = 1e-8

NC = 2
NS = 16
NW = NC * NS
EW = 5120
E_PAD = NW * EW
CH = 64
NCHUNK = EW // CH
N_PAD = 10240
ROWS_PER_TILE = N_PAD // NS
DRAIN_CH = CH
NDRAIN = ROWS_PER_TILE // DRAIN_CH

LOG2 = float(np.log(2.0))
F32 = jnp.float32


def _ssp(x):
    return jax.nn.softplus(x) - LOG2


_RD = 1024


def _desc_body(rbfs_ref, cut_ref, vec_ref, dist_ref, wf_ref, bf_ref,
               *out_refs):
    x = jnp.dot(rbfs_ref[...], wf_ref[...], preferred_element_type=F32)
    x = (x + bf_ref[...]) * cut_ref[...]
    dirs = vec_ref[...] / dist_ref[...]
    for b in range(NB):
        for j in range(3):
            out_refs[b * 3 + j][...] = x[:, (b * 3 + j) * F:(b * 3 + j + 1) * F]
    dirs_ref = out_refs[9]
    for k in range(3):
        dirs_ref[k, :, :] = jnp.broadcast_to(dirs[:, k:k + 1], (_RD, 16))


def _desc_call(rbfs_p, cut_p, vec_p, dist_p, wf, bf2):
    grid = (E_PAD // _RD,)
    d_specs = [pl.BlockSpec((_RD, F), lambda i: (i, 0)) for _ in range(9)]
    return pl.pallas_call(
        _desc_body,
        grid=grid,
        in_specs=[
            pl.BlockSpec((_RD, NRBF), lambda i: (i, 0)),
            pl.BlockSpec((_RD, 1), lambda i: (i, 0)),
            pl.BlockSpec((_RD, 3), lambda i: (i, 0)),
            pl.BlockSpec((_RD, 1), lambda i: (i, 0)),
            pl.BlockSpec((NRBF, NB * 3 * F), lambda i: (0, 0)),
            pl.BlockSpec((1, NB * 3 * F), lambda i: (0, 0)),
        ],
        out_specs=d_specs + [pl.BlockSpec((3, _RD, 16), lambda i: (0, i, 0))],
        out_shape=[jax.ShapeDtypeStruct((E_PAD, F), F32) for _ in range(9)]
        + [jax.ShapeDtypeStruct((3, E_PAD, 16), F32)],
    )(rbfs_p, cut_p, vec_p, dist_p, wf, bf2)


_RN = 1000


def _dense_body(q_ref, w1_ref, b1_ref, w2_ref, b2_ref, x0_ref, x1_ref, x2_ref):
    h = _ssp(jnp.dot(q_ref[...], w1_ref[...], preferred_element_type=F32)
             + b1_ref[...])
    x = jnp.dot(h, w2_ref[...], preferred_element_type=F32) + b2_ref[...]
    x0_ref[...] = x[:, :F]
    x1_ref[...] = x[:, F:2 * F]
    x2_ref[...] = x[:, 2 * F:]


def _dense_call(q, w1, b1, w2, b2):
    grid = (N // _RN,)
    return pl.pallas_call(
        _dense_body,
        grid=grid,
        in_specs=[
            pl.BlockSpec((_RN, F), lambda i: (i, 0)),
            pl.BlockSpec((F, F), lambda i: (0, 0)),
            pl.BlockSpec((1, F), lambda i: (0, 0)),
            pl.BlockSpec((F, 3 * F), lambda i: (0, 0)),
            pl.BlockSpec((1, 3 * F), lambda i: (0, 0)),
        ],
        out_specs=[pl.BlockSpec((_RN, F), lambda i: (i, 0))] * 3,
        out_shape=[jax.ShapeDtypeStruct((N, F), F32)] * 3,
    )(q, w1, b1, w2, b2)


def _mix_body(has_mu, mu_nat, *refs):
    if has_mu:
        (q_ref, mu0_ref, mu1_ref, mu2_ref, dqp_ref, dmup_ref,
         wmix_ref, wm1_ref, bm1_ref, wm2_ref, bm2_ref) = refs[:11]
        out_refs = refs[11:]
        mu_in = (mu0_ref, mu1_ref, mu2_ref)
    else:
        (q_ref, dqp_ref, dmup_ref,
         wmix_ref, wm1_ref, bm1_ref, wm2_ref, bm2_ref) = refs[:8]
        out_refs = refs[8:]
        mu_in = None
    qn = q_ref[...] + dqp_ref[0] + dqp_ref[1]
    mun, mu_V, mu_W = [], [], []
    for k in range(3):
        m = dmup_ref[0, k] + dmup_ref[1, k]
        if has_mu:
            m = m + mu_in[k][...]
        mun.append(m)
        mm = jnp.dot(m, wmix_ref[...], preferred_element_type=F32)
        mu_V.append(mm[:, :F])
        mu_W.append(mm[:, F:])
    vn2 = mu_V[0] ** 2 + mu_V[1] ** 2 + mu_V[2] ** 2
    ctx = jnp.concatenate([qn, jnp.sqrt(vn2 + EPS)], axis=1)
    h = _ssp(jnp.dot(ctx, wm1_ref[...], preferred_element_type=F32)
             + bm1_ref[...])
    xm = jnp.dot(h, wm2_ref[...], preferred_element_type=F32) + bm2_ref[...]
    dot = mu_V[0] * mu_W[0] + mu_V[1] * mu_W[1] + mu_V[2] * mu_W[2]
    out_refs[0][...] = qn + xm[:, :F] + xm[:, 2 * F:] * dot
    for k in range(3):
        mo = mun[k] + xm[:, F:2 * F] * mu_W[k]
        if mu_nat:
            out_refs[1][:, k, :] = mo
        else:
            out_refs[1 + k][...] = mo


def _mix_call(has_mu, mu_nat, q, mu, dqp, dmup, wmix, wm1, bm1, wm2, bm2):
    grid = (N // _RN,)
    row = pl.BlockSpec((_RN, F), lambda i: (i, 0))
    in_specs = [row]
    args = [q]
    if has_mu:
        in_specs += [row] * 3
        args += list(mu)
    in_specs += [
        pl.BlockSpec((NC, _RN, F), lambda i: (0, i, 0)),
        pl.BlockSpec((NC, 3, _RN, F), lambda i: (0, 0, i, 0)),
        pl.BlockSpec((F, 2 * F), lambda i: (0, 0)),
        pl.BlockSpec((2 * F, F), lambda i: (0, 0)),
        pl.BlockSpec((1, F), lambda i: (0, 0)),
        pl.BlockSpec((F, 3 * F), lambda i: (0, 0)),
        pl.BlockSpec((1, 3 * F), lambda i: (0, 0)),
    ]
    args += [dqp, dmup, wmix, wm1, bm1, wm2, bm2]
    if mu_nat:
        out_specs = [row, pl.BlockSpec((_RN, 3, F), lambda i: (i, 0, 0))]
        out_shape = [jax.ShapeDtypeStruct((N, F), F32),
                     jax.ShapeDtypeStruct((N, 3, F), F32)]
    else:
        out_specs = [row] * 4
        out_shape = [jax.ShapeDtypeStruct((N, F), F32)] * 4
    return pl.pallas_call(
        functools.partial(_mix_body, has_mu, mu_nat),
        grid=grid,
        in_specs=in_specs,
        out_specs=out_specs,
        out_shape=out_shape,
    )(*args)


def _sc_body(b0, *refs):
    if b0:
        (x0, x1, x2, d0, d1, d2, dirsb, idxi_h, idxj_h,
         dqp_o, dmup_o,
         acc, ba, bb, bc, dv, ixi, ixj) = refs
        mu = None
    else:
        (x0, x1, x2, mu0, mu1, mu2, d0, d1, d2, dirsb, idxi_h, idxj_h,
         dqp_o, dmup_o,
         acc, ba, bb, bc, dv, ixi, ixj) = refs
        mu = (mu0, mu1, mu2)
    c = lax.axis_index("c")
    s = lax.axis_index("s")
    wid = c * NS + s
    ebase = wid * EW
    rbase = s * ROWS_PER_TILE

    def zz(i, carry):
        ba[i // 8, pl.ds((i % 8) * 16, 16)] = jnp.zeros((16,), F32)
        return carry

    for p in range(4):
        lax.fori_loop(0, CH * 8, zz, 0)
        for d in range(NDRAIN):
            pltpu.sync_copy(ba, acc.at[pl.ds(rbase + d * DRAIN_CH, DRAIN_CH)])
        plsc.subcore_barrier()

        def chunk(i, carry, p=p):
            base = ebase + i * CH
            pltpu.sync_copy(idxi_h.at[pl.ds(base, CH)], ixi)
            pltpu.sync_copy(idxj_h.at[pl.ds(base, CH)], ixj)
            if p == 0:
                pltpu.sync_copy(x0.at[ixj], ba)
                pltpu.sync_copy(d0.at[pl.ds(base, CH)], bb)

                def ce(e, cc):
                    for g in range(8):
                        sl = pl.ds(g * 16, 16)
                        ba[e, sl] = ba[e, sl] * bb[e, sl]
                    return cc

                lax.fori_loop(0, CH, ce, 0)
            else:
                k = p - 1
                pltpu.sync_copy(dirsb.at[k].at[pl.ds(base, CH)], dv)
                if not b0:
                    pltpu.sync_copy(x2.at[ixj], ba)
                    pltpu.sync_copy(mu[k].at[ixj], bb)
                    pltpu.sync_copy(d2.at[pl.ds(base, CH)], bc)

                    def ce1(e, cc):
                        for g in range(8):
                            sl = pl.ds(g * 16, 16)
                            ba[e, sl] = ba[e, sl] * bb[e, sl] * bc[e, sl]
                        return cc

                    lax.fori_loop(0, CH, ce1, 0)
                pltpu.sync_copy(x1.at[ixj], bb)
                pltpu.sync_copy(d1.at[pl.ds(base, CH)], bc)

                def ce2(e, cc):
                    dvv = dv[e, pl.ds(0, 16)]
                    for g in range(8):
                        sl = pl.ds(g * 16, 16)
                        r = bb[e, sl] * bc[e, sl] * dvv
                        if not b0:
                            r = r + ba[e, sl]
                        ba[e, sl] = r
                    return cc

                lax.fori_loop(0, CH, ce2, 0)
            pltpu.sync_copy(ba, acc.at[ixi], add=True)
            return carry

        lax.fori_loop(0, NCHUNK, chunk, 0)
        plsc.subcore_barrier()

        for d in range(NDRAIN):
            r0 = rbase + d * DRAIN_CH
            pltpu.sync_copy(acc.at[pl.ds(r0, DRAIN_CH)], ba)
            if p == 0:
                pltpu.sync_copy(ba, dqp_o.at[c].at[pl.ds(r0, DRAIN_CH)])
            else:
                pltpu.sync_copy(
                    ba, dmup_o.at[c].at[p - 1].at[pl.ds(r0, DRAIN_CH)])


def _sc_call(b0, x, mu, d, dirsb, idxi_p, idxj_p):
    mesh = plsc.VectorSubcoreMesh(core_axis_name="c", subcore_axis_name="s",
                                  num_cores=NC, num_subcores=NS)
    out_type = (jax.ShapeDtypeStruct((NC, N_PAD, F), F32),
                jax.ShapeDtypeStruct((NC, 3, N_PAD, F), F32))
    scratch = [
        pltpu.VMEM_SHARED((N_PAD, F), F32),
        pltpu.VMEM((CH, F), F32),
        pltpu.VMEM((CH, F), F32),
        pltpu.VMEM((CH, F), F32),
        pltpu.VMEM((CH, 16), F32),
        pltpu.VMEM((CH,), jnp.int32),
        pltpu.VMEM((CH,), jnp.int32),
    ]
    kern = pl.kernel(functools.partial(_sc_body, b0),
                     out_type=out_type, mesh=mesh, scratch_types=scratch)
    if b0:
        return kern(x[0], x[1], x[2], d[0], d[1], d[2], dirsb, idxi_p, idxj_p)
    return kern(x[0], x[1], x[2], mu[0], mu[1], mu[2],
                d[0], d[1], d[2], dirsb, idxi_p, idxj_p)


def kernel(features, rbfs, distances, vectors, cutoff, idx_i, idx_j,
           Wf, bf, Wi1, bi1, Wi2, bi2, Wmix, Wm1, bm1, Wm2, bm2):
    pad = E_PAD - E
    rbfs_p = jnp.pad(rbfs, ((0, pad), (0, 0)))
    cut_p = jnp.pad(cutoff, (0, pad)).reshape(E_PAD, 1)
    vec_p = jnp.pad(vectors, ((0, pad), (0, 0)))
    dist_p = jnp.pad(distances, (0, pad), constant_values=1.0).reshape(E_PAD, 1)
    idxi_p = jnp.pad(idx_i, (0, pad))
    idxj_p = jnp.pad(idx_j, (0, pad))

    desc_out = _desc_call(rbfs_p, cut_p, vec_p, dist_p, Wf, bf.reshape(1, -1))
    dtabs = [desc_out[b * 3:(b + 1) * 3] for b in range(NB)]
    dirsb = desc_out[9]

    q = features
    mu = None
    mu_nat = None
    for b in range(NB):
        x = _dense_call(q, Wi1[b], bi1[b].reshape(1, F),
                        Wi2[b], bi2[b].reshape(1, 3 * F))
        dqp, dmup = _sc_call(b == 0, x, mu, dtabs[b], dirsb, idxi_p, idxj_p)
        last = b == NB - 1
        outs = _mix_call(b != 0, last, q, mu, dqp, dmup,
                         Wmix[b], Wm1[b], bm1[b].reshape(1, F),
                         Wm2[b], bm2[b].reshape(1, 3 * F))
        if last:
            q, mu_nat = outs
        else:
            q, mu = outs[0], outs[1:]
    return q, mu_nat


# speedup vs baseline: 8.1989x; 1.5617x over previous
import functools

import jax
import jax.numpy as jnp
import numpy as np
from jax import lax
from jax.experimental import pallas as pl
from jax.experimental.pallas import tpu as pltpu
from jax.experimental.pallas import tpu_sc as plsc

N = 10000
E = 160000
F = 128
NRBF = 20
NB = 3
EPS = 1e-8

NC = 2
NS = 16
NW = NC * NS
EW = 5120
E_PAD = NW * EW
CH = 64
NCHUNK = EW // CH
N_PAD = 10240
ROWS_PER_TILE = N_PAD // NS
DRAIN_CH = CH
NDRAIN = ROWS_PER_TILE // DRAIN_CH

LOG2 = float(np.log(2.0))
F32 = jnp.float32


def _ssp(x):
    return jax.nn.softplus(x) - LOG2


_RD = 1024


def _desc_body(rbfs_ref, cut_ref, vec_ref, dist_ref, wf_ref, bf_ref,
               *out_refs):
    x = jnp.dot(rbfs_ref[...], wf_ref[...], preferred_element_type=F32)
    x = (x + bf_ref[...]) * cut_ref[...]
    dirs = vec_ref[...] / dist_ref[...]
    for b in range(NB):
        d0 = x[:, (b * 3) * F:(b * 3 + 1) * F]
        d1 = x[:, (b * 3 + 1) * F:(b * 3 + 2) * F]
        d2 = x[:, (b * 3 + 2) * F:(b * 3 + 3) * F]
        out_refs[b * 5][...] = d0
        for k in range(3):
            out_refs[b * 5 + 1 + k][...] = d1 * dirs[:, k:k + 1]
        out_refs[b * 5 + 4][...] = d2


def _desc_call(rbfs_p, cut_p, vec_p, dist_p, wf, bf2):
    grid = (E_PAD // _RD,)
    d_specs = [pl.BlockSpec((_RD, F), lambda i: (i, 0)) for _ in range(15)]
    return pl.pallas_call(
        _desc_body,
        grid=grid,
        in_specs=[
            pl.BlockSpec((_RD, NRBF), lambda i: (i, 0)),
            pl.BlockSpec((_RD, 1), lambda i: (i, 0)),
            pl.BlockSpec((_RD, 3), lambda i: (i, 0)),
            pl.BlockSpec((_RD, 1), lambda i: (i, 0)),
            pl.BlockSpec((NRBF, NB * 3 * F), lambda i: (0, 0)),
            pl.BlockSpec((1, NB * 3 * F), lambda i: (0, 0)),
        ],
        out_specs=d_specs,
        out_shape=[jax.ShapeDtypeStruct((E_PAD, F), F32) for _ in range(15)],
    )(rbfs_p, cut_p, vec_p, dist_p, wf, bf2)


_RN = 1000


def _dense_body(q_ref, w1_ref, b1_ref, w2_ref, b2_ref, x0_ref, x1_ref, x2_ref):
    h = _ssp(jnp.dot(q_ref[...], w1_ref[...], preferred_element_type=F32)
             + b1_ref[...])
    x = jnp.dot(h, w2_ref[...], preferred_element_type=F32) + b2_ref[...]
    x0_ref[...] = x[:, :F]
    x1_ref[...] = x[:, F:2 * F]
    x2_ref[...] = x[:, 2 * F:]


def _dense_call(q, w1, b1, w2, b2):
    grid = (N // _RN,)
    return pl.pallas_call(
        _dense_body,
        grid=grid,
        in_specs=[
            pl.BlockSpec((_RN, F), lambda i: (i, 0)),
            pl.BlockSpec((F, F), lambda i: (0, 0)),
            pl.BlockSpec((1, F), lambda i: (0, 0)),
            pl.BlockSpec((F, 3 * F), lambda i: (0, 0)),
            pl.BlockSpec((1, 3 * F), lambda i: (0, 0)),
        ],
        out_specs=[pl.BlockSpec((_RN, F), lambda i: (i, 0))] * 3,
        out_shape=[jax.ShapeDtypeStruct((N, F), F32)] * 3,
    )(q, w1, b1, w2, b2)


def _mix_body(has_mu, mu_nat, *refs):
    if has_mu:
        (q_ref, mu0_ref, mu1_ref, mu2_ref, dqp_ref, dmup_ref,
         wmix_ref, wm1_ref, bm1_ref, wm2_ref, bm2_ref) = refs[:11]
        out_refs = refs[11:]
        mu_in = (mu0_ref, mu1_ref, mu2_ref)
    else:
        (q_ref, dqp_ref, dmup_ref,
         wmix_ref, wm1_ref, bm1_ref, wm2_ref, bm2_ref) = refs[:8]
        out_refs = refs[8:]
        mu_in = None
    qn = q_ref[...] + dqp_ref[0] + dqp_ref[1]
    mun, mu_V, mu_W = [], [], []
    for k in range(3):
        m = dmup_ref[0, k] + dmup_ref[1, k]
        if has_mu:
            m = m + mu_in[k][...]
        mun.append(m)
        mm = jnp.dot(m, wmix_ref[...], preferred_element_type=F32)
        mu_V.append(mm[:, :F])
        mu_W.append(mm[:, F:])
    vn2 = mu_V[0] ** 2 + mu_V[1] ** 2 + mu_V[2] ** 2
    ctx = jnp.concatenate([qn, jnp.sqrt(vn2 + EPS)], axis=1)
    h = _ssp(jnp.dot(ctx, wm1_ref[...], preferred_element_type=F32)
             + bm1_ref[...])
    xm = jnp.dot(h, wm2_ref[...], preferred_element_type=F32) + bm2_ref[...]
    dot = mu_V[0] * mu_W[0] + mu_V[1] * mu_W[1] + mu_V[2] * mu_W[2]
    out_refs[0][...] = qn + xm[:, :F] + xm[:, 2 * F:] * dot
    for k in range(3):
        mo = mun[k] + xm[:, F:2 * F] * mu_W[k]
        if mu_nat:
            out_refs[1][:, k, :] = mo
        else:
            out_refs[1 + k][...] = mo


def _mix_call(has_mu, mu_nat, q, mu, dqp, dmup, wmix, wm1, bm1, wm2, bm2):
    grid = (N // _RN,)
    row = pl.BlockSpec((_RN, F), lambda i: (i, 0))
    in_specs = [row]
    args = [q]
    if has_mu:
        in_specs += [row] * 3
        args += list(mu)
    in_specs += [
        pl.BlockSpec((NC, _RN, F), lambda i: (0, i, 0)),
        pl.BlockSpec((NC, 3, _RN, F), lambda i: (0, 0, i, 0)),
        pl.BlockSpec((F, 2 * F), lambda i: (0, 0)),
        pl.BlockSpec((2 * F, F), lambda i: (0, 0)),
        pl.BlockSpec((1, F), lambda i: (0, 0)),
        pl.BlockSpec((F, 3 * F), lambda i: (0, 0)),
        pl.BlockSpec((1, 3 * F), lambda i: (0, 0)),
    ]
    args += [dqp, dmup, wmix, wm1, bm1, wm2, bm2]
    if mu_nat:
        out_specs = [row, pl.BlockSpec((_RN, 3, F), lambda i: (i, 0, 0))]
        out_shape = [jax.ShapeDtypeStruct((N, F), F32),
                     jax.ShapeDtypeStruct((N, 3, F), F32)]
    else:
        out_specs = [row] * 4
        out_shape = [jax.ShapeDtypeStruct((N, F), F32)] * 4
    return pl.pallas_call(
        functools.partial(_mix_body, has_mu, mu_nat),
        grid=grid,
        in_specs=in_specs,
        out_specs=out_specs,
        out_shape=out_shape,
    )(*args)


def _sc_body(b0, *refs):
    if b0:
        (x0, x1, x2, d0, dx0, dx1, dx2, d2, idxi_h, idxj_h,
         dqp_o, dmup_o,
         acc, ba, bb, bc, bd, be, ixi, ixj, sem) = refs
        mu = None
    else:
        (x0, x1, x2, mu0, mu1, mu2, d0, dx0, dx1, dx2, d2, idxi_h, idxj_h,
         dqp_o, dmup_o,
         acc, ba, bb, bc, bd, be, ixi, ixj, sem) = refs
        mu = (mu0, mu1, mu2)
    dx = (dx0, dx1, dx2)
    c = lax.axis_index("c")
    s = lax.axis_index("s")
    wid = c * NS + s
    ebase = wid * EW
    rbase = s * ROWS_PER_TILE

    def zz(i, carry):
        ba[i // 8, pl.ds((i % 8) * 16, 16)] = jnp.zeros((16,), F32)
        return carry

    for p in range(4):
        lax.fori_loop(0, CH * 8, zz, 0)
        for d in range(NDRAIN):
            pltpu.sync_copy(ba, acc.at[pl.ds(rbase + d * DRAIN_CH, DRAIN_CH)])
        plsc.subcore_barrier()

        def chunk(i, carry, p=p):
            base = ebase + i * CH
            ci = pltpu.async_copy(idxi_h.at[pl.ds(base, CH)], ixi, sem)
            cj = pltpu.async_copy(idxj_h.at[pl.ds(base, CH)], ixj, sem)
            ci.wait()
            cj.wait()
            waits = []
            if p == 0:
                waits.append(pltpu.async_copy(x0.at[ixj], ba, sem))
                waits.append(pltpu.async_copy(d0.at[pl.ds(base, CH)], bb, sem))
            else:
                k = p - 1
                waits.append(pltpu.async_copy(x1.at[ixj], bd, sem))
                waits.append(pltpu.async_copy(
                    dx[k].at[pl.ds(base, CH)], be, sem))
                if not b0:
                    waits.append(pltpu.async_copy(x2.at[ixj], ba, sem))
                    waits.append(pltpu.async_copy(mu[k].at[ixj], bb, sem))
                    waits.append(pltpu.async_copy(d2.at[pl.ds(base, CH)], bc, sem))
            for w in waits:
                w.wait()
            if p == 0:
                def ce(e, cc):
                    for g in range(8):
                        sl = pl.ds(g * 16, 16)
                        ba[e, sl] = ba[e, sl] * bb[e, sl]
                    return cc

                lax.fori_loop(0, CH, ce, 0)
            else:
                def ce(e, cc):
                    for g in range(8):
                        sl = pl.ds(g * 16, 16)
                        r = bd[e, sl] * be[e, sl]
                        if not b0:
                            r = r + ba[e, sl] * bb[e, sl] * bc[e, sl]
                        ba[e, sl] = r
                    return cc

                lax.fori_loop(0, CH, ce, 0)
            pltpu.sync_copy(ba, acc.at[ixi], add=True)
            return carry

        lax.fori_loop(0, NCHUNK, chunk, 0)
        plsc.subcore_barrier()

        for d in range(NDRAIN):
            r0 = rbase + d * DRAIN_CH
            pltpu.sync_copy(acc.at[pl.ds(r0, DRAIN_CH)], ba)
            if p == 0:
                pltpu.sync_copy(ba, dqp_o.at[c].at[pl.ds(r0, DRAIN_CH)])
            else:
                pltpu.sync_copy(
                    ba, dmup_o.at[c].at[p - 1].at[pl.ds(r0, DRAIN_CH)])


def _sc_call(b0, x, mu, d, idxi_p, idxj_p):
    mesh = plsc.VectorSubcoreMesh(core_axis_name="c", subcore_axis_name="s",
                                  num_cores=NC, num_subcores=NS)
    out_type = (jax.ShapeDtypeStruct((NC, N_PAD, F), F32),
                jax.ShapeDtypeStruct((NC, 3, N_PAD, F), F32))
    scratch = [
        pltpu.VMEM_SHARED((N_PAD, F), F32),
        pltpu.VMEM((CH, F), F32),
        pltpu.VMEM((CH, F), F32),
        pltpu.VMEM((CH, F), F32),
        pltpu.VMEM((CH, F), F32),
        pltpu.VMEM((CH, F), F32),
        pltpu.VMEM((CH,), jnp.int32),
        pltpu.VMEM((CH,), jnp.int32),
        pltpu.SemaphoreType.DMA,
    ]
    kern = pl.kernel(functools.partial(_sc_body, b0),
                     out_type=out_type, mesh=mesh, scratch_types=scratch)
    if b0:
        return kern(x[0], x[1], x[2], *d, idxi_p, idxj_p)
    return kern(x[0], x[1], x[2], mu[0], mu[1], mu[2], *d, idxi_p, idxj_p)


def kernel(features, rbfs, distances, vectors, cutoff, idx_i, idx_j,
           Wf, bf, Wi1, bi1, Wi2, bi2, Wmix, Wm1, bm1, Wm2, bm2):
    pad = E_PAD - E
    rbfs_p = jnp.pad(rbfs, ((0, pad), (0, 0)))
    cut_p = jnp.pad(cutoff, (0, pad)).reshape(E_PAD, 1)
    vec_p = jnp.pad(vectors, ((0, pad), (0, 0)))
    dist_p = jnp.pad(distances, (0, pad), constant_values=1.0).reshape(E_PAD, 1)
    idxi_p = jnp.pad(idx_i, (0, pad))
    idxj_p = jnp.pad(idx_j, (0, pad))

    desc_out = _desc_call(rbfs_p, cut_p, vec_p, dist_p, Wf, bf.reshape(1, -1))
    dtabs = [desc_out[b * 5:(b + 1) * 5] for b in range(NB)]

    q = features
    mu = None
    mu_nat = None
    for b in range(NB):
        x = _dense_call(q, Wi1[b], bi1[b].reshape(1, F),
                        Wi2[b], bi2[b].reshape(1, 3 * F))
        dqp, dmup = _sc_call(b == 0, x, mu, dtabs[b], idxi_p, idxj_p)
        last = b == NB - 1
        outs = _mix_call(b != 0, last, q, mu, dqp, dmup,
                         Wmix[b], Wm1[b], bm1[b].reshape(1, F),
                         Wm2[b], bm2[b].reshape(1, 3 * F))
        if last:
            q, mu_nat = outs
        else:
            q, mu = outs[0], outs[1:]
    return q, mu_nat
